# baseline (speedup 1.0000x reference)
import functools

import jax
import jax.numpy as jnp
from jax import lax
from jax.experimental import pallas as pl
from jax.experimental.pallas import tpu as pltpu
from jax.experimental.pallas import tpu_sc as plsc

_N = 10000
_E = 160000
_CW = 128
_NW = 32
_BE = 64
_EPAD = 163840
_NBT = _EPAD // (16 * _BE)
_NB0 = 96
_NB1 = _NBT - _NB0
_NBMAX = max(_NB0, _NB1)
_NACC = 10112
_ZR = _NACC // 16
_ZB = 32
_BN = 1000


def _sc_segment_sum(C, with_deg=False):
    mesh = plsc.VectorSubcoreMesh(core_axis_name="c", subcore_axis_name="s")
    CT = C + (1 if with_deg else 0)
    out_type = [jax.ShapeDtypeStruct((2, CT, _NACC, _CW), jnp.float32)]
    scratch = [
        pltpu.VMEM((_NBMAX, _BE), jnp.int32),
        pltpu.VMEM((_NBMAX, _BE), jnp.int32),
        pltpu.VMEM((2, _BE, _CW), jnp.float32),
        pltpu.VMEM((_ZB, _CW), jnp.float32),
        pltpu.VMEM_SHARED((_NACC, _CW), jnp.float32),
        pltpu.SemaphoreType.DMA,
        pltpu.SemaphoreType.DMA,
    ]

    def body(*refs):
        h = refs[:C]
        srcm, dstm = refs[C:C + 2]
        i = C + 2
        if with_deg:
            ones_h = refs[i]
            i += 1
        out = refs[i]
        src_v, dst_v, rows_v, zero_v, acc, gsem, ssem = refs[i + 1:i + 8]

        cid = lax.axis_index("c")
        sid = lax.axis_index("s")
        wid = cid * 16 + sid
        nb = _NB0 + cid * (_NB1 - _NB0)
        nbp = _NB0 // 2 + cid * (_NB1 // 2 - _NB0 // 2)

        pltpu.sync_copy(srcm.at[wid], src_v)
        pltpu.sync_copy(dstm.at[wid], dst_v)

        def zfill(j, carry):
            zero_v[j // 8, pl.ds((j % 8) * 16, 16)] = jnp.zeros(
                (16,), jnp.float32)
            return carry

        lax.fori_loop(0, _ZB * 8, zfill, 0)

        for ch in range(CT):
            deg_pass = ch == C
            for r in range(_ZR // _ZB):
                pltpu.sync_copy(
                    zero_v, acc.at[pl.ds(sid * _ZR + r * _ZB, _ZB)])
            if _ZR % _ZB:
                pltpu.sync_copy(
                    zero_v.at[pl.ds(0, _ZR % _ZB)],
                    acc.at[pl.ds(sid * _ZR + (_ZR // _ZB) * _ZB, _ZR % _ZB)])
            if deg_pass:
                pltpu.sync_copy(ones_h, rows_v.at[0])
            plsc.subcore_barrier()

            def sstart(b, buf):
                pltpu.async_copy(rows_v.at[buf], acc.at[dst_v.at[b]], ssem,
                                 add=True)

            def swait(buf):
                pltpu.make_async_copy(
                    rows_v.at[buf], acc.at[dst_v.at[0]], ssem).wait()

            if deg_pass:
                def deg_body(b, carry):
                    pltpu.sync_copy(rows_v.at[0], acc.at[dst_v.at[b]],
                                    add=True)
                    return carry

                lax.fori_loop(0, nb, deg_body, 0)
            else:
                def gstart(b, buf, ch=ch):
                    pltpu.async_copy(h[ch].at[src_v.at[b]], rows_v.at[buf],
                                     gsem)

                def gwait(buf, ch=ch):
                    pltpu.make_async_copy(
                        h[ch].at[src_v.at[0]], rows_v.at[buf], gsem).wait()

                gstart(0, 0)
                gstart(1, 1)

                def loop_body(i2, carry):
                    b0 = i2 * 2
                    b1 = b0 + 1
                    gwait(0)
                    sstart(b0, 0)
                    gwait(1)
                    sstart(b1, 1)
                    swait(0)

                    @pl.when(b0 + 2 < nb)
                    def _():
                        gstart(b0 + 2, 0)

                    swait(1)

                    @pl.when(b1 + 2 < nb)
                    def _():
                        gstart(b1 + 2, 1)

                    return carry

                lax.fori_loop(0, nbp, loop_body, 0)

            plsc.subcore_barrier()
            pltpu.sync_copy(acc.at[pl.ds(sid * _ZR, _ZR)],
                            out.at[cid, ch, pl.ds(sid * _ZR, _ZR)])
            plsc.subcore_barrier()

    return pl.kernel(body, out_type=out_type, mesh=mesh, scratch_types=scratch)


def _tc_sage_layer(P, deg, x_chunks, Wl, b, Wr, relu, cout):
    cin = len(x_chunks)
    dout = Wl.shape[1]
    grid = (_N // _BN,)

    def body(P_ref, deg_ref, *rest):
        x_refs = rest[:cin]
        Wl_ref, b_ref, Wr_ref = rest[cin:cin + 3]
        out_refs = rest[cin + 3:]
        d = deg_ref[0, :, 0:1] + deg_ref[1, :, 0:1]
        invd = 1.0 / jnp.maximum(d, 1.0)
        acc = jnp.zeros((_BN, dout), jnp.float32)
        for c in range(cin):
            aggc = (P_ref[0, c] + P_ref[1, c]) * invd
            acc = acc + jnp.dot(aggc, Wl_ref[c * _CW:(c + 1) * _CW, :],
                                preferred_element_type=jnp.float32)
            acc = acc + jnp.dot(x_refs[c][...], Wr_ref[c * _CW:(c + 1) * _CW, :],
                                preferred_element_type=jnp.float32)
        acc = acc + b_ref[...]
        if relu:
            acc = jnp.maximum(acc, 0.0)
        for c in range(cout):
            out_refs[c][...] = acc[:, c * _CW:(c + 1) * _CW]

    in_specs = [
        pl.BlockSpec((2, cin, _BN, _CW), lambda i: (0, 0, i, 0)),
        pl.BlockSpec((2, _BN, _CW), lambda i: (0, i, 0)),
    ]
    in_specs += [pl.BlockSpec((_BN, _CW), lambda i: (i, 0)) for _ in range(cin)]
    in_specs += [
        pl.BlockSpec((cin * _CW, dout), lambda i: (0, 0)),
        pl.BlockSpec((1, dout), lambda i: (0, 0)),
        pl.BlockSpec((cin * _CW, dout), lambda i: (0, 0)),
    ]
    out_specs = [pl.BlockSpec((_BN, _CW), lambda i: (i, 0)) for _ in range(cout)]
    out_shape = [jax.ShapeDtypeStruct((_N, _CW), jnp.float32) for _ in range(cout)]
    return pl.pallas_call(body, grid=grid, in_specs=in_specs,
                          out_specs=out_specs, out_shape=out_shape)(
        P, deg, *x_chunks, Wl, b, Wr)


def _tc_transform3(x_chunks, W3l, W3r, b3):
    cin = len(x_chunks)
    dout = W3l.shape[1]
    czo = dout // _CW
    grid = (_N // _BN,)

    def body(*refs):
        x_refs = refs[:cin]
        Wl_ref, Wr_ref, b_ref = refs[cin:cin + 3]
        z_refs = refs[cin + 3:cin + 3 + czo]
        r_ref = refs[cin + 3 + czo]
        accz = jnp.zeros((_BN, dout), jnp.float32)
        accr = jnp.zeros((_BN, dout), jnp.float32)
        for c in range(cin):
            xc = x_refs[c][...]
            accz = accz + jnp.dot(xc, Wl_ref[c * _CW:(c + 1) * _CW, :],
                                  preferred_element_type=jnp.float32)
            accr = accr + jnp.dot(xc, Wr_ref[c * _CW:(c + 1) * _CW, :],
                                  preferred_element_type=jnp.float32)
        accr = accr + b_ref[...]
        for c in range(czo):
            z_refs[c][...] = accz[:, c * _CW:(c + 1) * _CW]
        r_ref[...] = accr

    in_specs = [pl.BlockSpec((_BN, _CW), lambda i: (i, 0)) for _ in range(cin)]
    in_specs += [
        pl.BlockSpec((cin * _CW, dout), lambda i: (0, 0)),
        pl.BlockSpec((cin * _CW, dout), lambda i: (0, 0)),
        pl.BlockSpec((1, dout), lambda i: (0, 0)),
    ]
    out_specs = [pl.BlockSpec((_BN, _CW), lambda i: (i, 0)) for _ in range(czo)]
    out_specs += [pl.BlockSpec((_BN, dout), lambda i: (i, 0))]
    out_shape = [jax.ShapeDtypeStruct((_N, _CW), jnp.float32) for _ in range(czo)]
    out_shape += [jax.ShapeDtypeStruct((_N, dout), jnp.float32)]
    outs = pl.pallas_call(body, grid=grid, in_specs=in_specs,
                          out_specs=out_specs, out_shape=out_shape)(
        *x_chunks, W3l, W3r, b3)
    return outs[:czo], outs[czo]


def _tc_combine3(P, deg, r):
    czo = P.shape[1]
    dout = r.shape[1]
    grid = (_N // _BN,)

    def body(P_ref, deg_ref, r_ref, out_ref):
        d = deg_ref[0, :, 0:1] + deg_ref[1, :, 0:1]
        invd = 1.0 / jnp.maximum(d, 1.0)
        for c in range(czo):
            out_ref[:, c * _CW:(c + 1) * _CW] = (
                (P_ref[0, c] + P_ref[1, c]) * invd
                + r_ref[:, c * _CW:(c + 1) * _CW])

    in_specs = [
        pl.BlockSpec((2, czo, _BN, _CW), lambda i: (0, 0, i, 0)),
        pl.BlockSpec((2, _BN, _CW), lambda i: (0, i, 0)),
        pl.BlockSpec((_BN, dout), lambda i: (i, 0)),
    ]
    out_specs = pl.BlockSpec((_BN, dout), lambda i: (i, 0))
    out_shape = jax.ShapeDtypeStruct((_N, dout), jnp.float32)
    return pl.pallas_call(body, grid=grid, in_specs=in_specs,
                          out_specs=out_specs, out_shape=out_shape)(P, deg, r)


def kernel(x, edge_index, W1l, b1, W1r, W2l, b2, W2r, W3l, b3, W3r):
    src = edge_index[0].astype(jnp.int32)
    dst = edge_index[1].astype(jnp.int32)
    pad = _EPAD - _E
    srcf = jnp.concatenate([src, jnp.zeros((pad,), jnp.int32)]).reshape(
        _EPAD // _BE, _BE)
    dstf = jnp.concatenate([dst, jnp.full((pad,), _N, jnp.int32)]).reshape(
        _EPAD // _BE, _BE)
    w = jnp.arange(_NW)
    wstart = jnp.where(w < 16, w * _NB0, 16 * _NB0 + (w - 16) * _NB1)
    nb_w = jnp.where(w < 16, _NB0, _NB1)
    bidx = wstart[:, None] + jnp.arange(_NBMAX)[None, :]
    valid = (jnp.arange(_NBMAX)[None, :] < nb_w[:, None])[:, :, None]
    bidx = jnp.minimum(bidx, _EPAD // _BE - 1)
    srcm = jnp.where(valid, srcf[bidx], 0)
    dstm = jnp.where(valid, dstf[bidx], _N)
    ones = jnp.ones((_BE, _CW), jnp.float32)

    x_ch = [x[:, c * _CW:(c + 1) * _CW] for c in range(2)]
    (P1,) = _sc_segment_sum(2, with_deg=True)(*x_ch, srcm, dstm, ones)
    degp = P1[:, 2]
    h1 = _tc_sage_layer(P1, degp, x_ch, W1l, b1.reshape(1, -1), W1r,
                        relu=True, cout=4)
    (P2,) = _sc_segment_sum(4)(*h1, srcm, dstm)
    h2 = _tc_sage_layer(P2, degp, h1, W2l, b2.reshape(1, -1), W2r,
                        relu=True, cout=4)
    z_ch, r3 = _tc_transform3(h2, W3l, W3r, b3.reshape(1, -1))
    (P3,) = _sc_segment_sum(2)(*z_ch, srcm, dstm)
    return _tc_combine3(P3, degp, r3)


# speedup vs baseline: 1.1030x; 1.1030x over previous
import functools

import jax
import jax.numpy as jnp
from jax import lax
from jax.experimental import pallas as pl
from jax.experimental.pallas import tpu as pltpu
from jax.experimental.pallas import tpu_sc as plsc

_N = 10000
_E = 160000
_CW = 128
_NW = 32
_BE = 64
_EPAD = 163840
_NBT = _EPAD // (16 * _BE)
_NB0 = 128
_NB1 = _NBT - _NB0
_NBMAX = max(_NB0, _NB1)
_NACC = 10112
_ZR = _NACC // 16
_ZB = 32
_BN = 1000


def _sc_segment_sum(C, with_deg=False):
    mesh = plsc.VectorSubcoreMesh(core_axis_name="c", subcore_axis_name="s")
    CT = C + (1 if with_deg else 0)
    out_type = [jax.ShapeDtypeStruct((2, CT, _NACC, _CW), jnp.float32)]
    scratch = [
        pltpu.VMEM((_NBMAX, _BE), jnp.int32),
        pltpu.VMEM((_NBMAX, _BE), jnp.int32),
        pltpu.VMEM((2, _BE, _CW), jnp.float32),
        pltpu.VMEM_SHARED((_NACC, _CW), jnp.float32),
        pltpu.SemaphoreType.DMA,
        pltpu.SemaphoreType.DMA,
    ]

    def body(*refs):
        h = refs[:C]
        srcm, dstm, zacc = refs[C:C + 3]
        i = C + 3
        if with_deg:
            ones_h = refs[i]
            i += 1
        out = refs[i]
        src_v, dst_v, rows_v, acc, gsem, ssem = refs[i + 1:i + 7]

        cid = lax.axis_index("c")
        sid = lax.axis_index("s")
        wid = cid * 16 + sid
        nb = _NB0 + cid * (_NB1 - _NB0)
        nbp = _NB0 // 2 + cid * (_NB1 // 2 - _NB0 // 2)

        pltpu.sync_copy(srcm.at[wid], src_v)
        pltpu.sync_copy(dstm.at[wid], dst_v)

        for ch in range(CT):
            deg_pass = ch == C
            pltpu.sync_copy(zacc, acc.at[pl.ds(sid * _ZR, _ZR)])
            if deg_pass:
                pltpu.sync_copy(ones_h, rows_v.at[0])
            plsc.subcore_barrier()

            def sstart(b, buf):
                pltpu.async_copy(rows_v.at[buf], acc.at[dst_v.at[b]], ssem,
                                 add=True)

            def swait(buf):
                pltpu.make_async_copy(
                    rows_v.at[buf], acc.at[dst_v.at[0]], ssem).wait()

            if deg_pass:
                def deg_body(b, carry):
                    pltpu.sync_copy(rows_v.at[0], acc.at[dst_v.at[b]],
                                    add=True)
                    return carry

                lax.fori_loop(0, nb, deg_body, 0)
            else:
                def gstart(b, buf, ch=ch):
                    pltpu.async_copy(h[ch].at[src_v.at[b]], rows_v.at[buf],
                                     gsem)

                def gwait(buf, ch=ch):
                    pltpu.make_async_copy(
                        h[ch].at[src_v.at[0]], rows_v.at[buf], gsem).wait()

                gstart(0, 0)
                gstart(1, 1)

                def loop_body(i2, carry):
                    b0 = i2 * 2
                    b1 = b0 + 1
                    gwait(0)
                    sstart(b0, 0)
                    gwait(1)
                    sstart(b1, 1)
                    swait(0)

                    @pl.when(b0 + 2 < nb)
                    def _():
                        gstart(b0 + 2, 0)

                    swait(1)

                    @pl.when(b1 + 2 < nb)
                    def _():
                        gstart(b1 + 2, 1)

                    return carry

                lax.fori_loop(0, nbp, loop_body, 0)

            plsc.subcore_barrier()
            pltpu.sync_copy(acc.at[pl.ds(sid * _ZR, _ZR)],
                            out.at[cid, ch, pl.ds(sid * _ZR, _ZR)])
            plsc.subcore_barrier()

    return pl.kernel(body, out_type=out_type, mesh=mesh, scratch_types=scratch)


def _tc_sage_layer(P, deg, x_chunks, Wl, b, Wr, relu, cout):
    cin = len(x_chunks)
    dout = Wl.shape[1]
    grid = (_N // _BN,)

    def body(P_ref, deg_ref, *rest):
        x_refs = rest[:cin]
        Wl_ref, b_ref, Wr_ref = rest[cin:cin + 3]
        out_refs = rest[cin + 3:]
        d = deg_ref[0, :, 0:1] + deg_ref[1, :, 0:1]
        invd = 1.0 / jnp.maximum(d, 1.0)
        acc = jnp.zeros((_BN, dout), jnp.float32)
        for c in range(cin):
            aggc = (P_ref[0, c] + P_ref[1, c]) * invd
            acc = acc + jnp.dot(aggc, Wl_ref[c * _CW:(c + 1) * _CW, :],
                                preferred_element_type=jnp.float32)
            acc = acc + jnp.dot(x_refs[c][...], Wr_ref[c * _CW:(c + 1) * _CW, :],
                                preferred_element_type=jnp.float32)
        acc = acc + b_ref[...]
        if relu:
            acc = jnp.maximum(acc, 0.0)
        for c in range(cout):
            out_refs[c][...] = acc[:, c * _CW:(c + 1) * _CW]

    in_specs = [
        pl.BlockSpec((2, cin, _BN, _CW), lambda i: (0, 0, i, 0)),
        pl.BlockSpec((2, _BN, _CW), lambda i: (0, i, 0)),
    ]
    in_specs += [pl.BlockSpec((_BN, _CW), lambda i: (i, 0)) for _ in range(cin)]
    in_specs += [
        pl.BlockSpec((cin * _CW, dout), lambda i: (0, 0)),
        pl.BlockSpec((1, dout), lambda i: (0, 0)),
        pl.BlockSpec((cin * _CW, dout), lambda i: (0, 0)),
    ]
    out_specs = [pl.BlockSpec((_BN, _CW), lambda i: (i, 0)) for _ in range(cout)]
    out_shape = [jax.ShapeDtypeStruct((_N, _CW), jnp.float32) for _ in range(cout)]
    return pl.pallas_call(body, grid=grid, in_specs=in_specs,
                          out_specs=out_specs, out_shape=out_shape)(
        P, deg, *x_chunks, Wl, b, Wr)


def _tc_transform3(x_chunks, W3l, W3r, b3):
    cin = len(x_chunks)
    dout = W3l.shape[1]
    czo = dout // _CW
    grid = (_N // _BN,)

    def body(*refs):
        x_refs = refs[:cin]
        Wl_ref, Wr_ref, b_ref = refs[cin:cin + 3]
        z_refs = refs[cin + 3:cin + 3 + czo]
        r_ref = refs[cin + 3 + czo]
        accz = jnp.zeros((_BN, dout), jnp.float32)
        accr = jnp.zeros((_BN, dout), jnp.float32)
        for c in range(cin):
            xc = x_refs[c][...]
            accz = accz + jnp.dot(xc, Wl_ref[c * _CW:(c + 1) * _CW, :],
                                  preferred_element_type=jnp.float32)
            accr = accr + jnp.dot(xc, Wr_ref[c * _CW:(c + 1) * _CW, :],
                                  preferred_element_type=jnp.float32)
        accr = accr + b_ref[...]
        for c in range(czo):
            z_refs[c][...] = accz[:, c * _CW:(c + 1) * _CW]
        r_ref[...] = accr

    in_specs = [pl.BlockSpec((_BN, _CW), lambda i: (i, 0)) for _ in range(cin)]
    in_specs += [
        pl.BlockSpec((cin * _CW, dout), lambda i: (0, 0)),
        pl.BlockSpec((cin * _CW, dout), lambda i: (0, 0)),
        pl.BlockSpec((1, dout), lambda i: (0, 0)),
    ]
    out_specs = [pl.BlockSpec((_BN, _CW), lambda i: (i, 0)) for _ in range(czo)]
    out_specs += [pl.BlockSpec((_BN, dout), lambda i: (i, 0))]
    out_shape = [jax.ShapeDtypeStruct((_N, _CW), jnp.float32) for _ in range(czo)]
    out_shape += [jax.ShapeDtypeStruct((_N, dout), jnp.float32)]
    outs = pl.pallas_call(body, grid=grid, in_specs=in_specs,
                          out_specs=out_specs, out_shape=out_shape)(
        *x_chunks, W3l, W3r, b3)
    return outs[:czo], outs[czo]


def _tc_combine3(P, deg, r):
    czo = P.shape[1]
    dout = r.shape[1]
    grid = (_N // _BN,)

    def body(P_ref, deg_ref, r_ref, out_ref):
        d = deg_ref[0, :, 0:1] + deg_ref[1, :, 0:1]
        invd = 1.0 / jnp.maximum(d, 1.0)
        for c in range(czo):
            out_ref[:, c * _CW:(c + 1) * _CW] = (
                (P_ref[0, c] + P_ref[1, c]) * invd
                + r_ref[:, c * _CW:(c + 1) * _CW])

    in_specs = [
        pl.BlockSpec((2, czo, _BN, _CW), lambda i: (0, 0, i, 0)),
        pl.BlockSpec((2, _BN, _CW), lambda i: (0, i, 0)),
        pl.BlockSpec((_BN, dout), lambda i: (i, 0)),
    ]
    out_specs = pl.BlockSpec((_BN, dout), lambda i: (i, 0))
    out_shape = jax.ShapeDtypeStruct((_N, dout), jnp.float32)
    return pl.pallas_call(body, grid=grid, in_specs=in_specs,
                          out_specs=out_specs, out_shape=out_shape)(P, deg, r)


def kernel(x, edge_index, W1l, b1, W1r, W2l, b2, W2r, W3l, b3, W3r):
    src = edge_index[0].astype(jnp.int32)
    dst = edge_index[1].astype(jnp.int32)
    pad = _EPAD - _E
    srcf = jnp.concatenate([src, jnp.zeros((pad,), jnp.int32)]).reshape(
        _EPAD // _BE, _BE)
    dstf = jnp.concatenate([dst, jnp.full((pad,), _N, jnp.int32)]).reshape(
        _EPAD // _BE, _BE)
    w = jnp.arange(_NW)
    wstart = jnp.where(w < 16, w * _NB0, 16 * _NB0 + (w - 16) * _NB1)
    nb_w = jnp.where(w < 16, _NB0, _NB1)
    bidx = wstart[:, None] + jnp.arange(_NBMAX)[None, :]
    valid = (jnp.arange(_NBMAX)[None, :] < nb_w[:, None])[:, :, None]
    bidx = jnp.minimum(bidx, _EPAD // _BE - 1)
    srcm = jnp.where(valid, srcf[bidx], 0)
    dstm = jnp.where(valid, dstf[bidx], _N)
    zacc = jnp.zeros((_ZR, _CW), jnp.float32)
    ones = jnp.ones((_BE, _CW), jnp.float32)

    x_ch = [x[:, c * _CW:(c + 1) * _CW] for c in range(2)]
    (P1,) = _sc_segment_sum(2, with_deg=True)(*x_ch, srcm, dstm, zacc, ones)
    degp = P1[:, 2]
    h1 = _tc_sage_layer(P1, degp, x_ch, W1l, b1.reshape(1, -1), W1r,
                        relu=True, cout=4)
    (P2,) = _sc_segment_sum(4)(*h1, srcm, dstm, zacc)
    h2 = _tc_sage_layer(P2, degp, h1, W2l, b2.reshape(1, -1), W2r,
                        relu=True, cout=4)
    z_ch, r3 = _tc_transform3(h2, W3l, W3r, b3.reshape(1, -1))
    (P3,) = _sc_segment_sum(2)(*z_ch, srcm, dstm, zacc)
    return _tc_combine3(P3, degp, r3)
